# Initial kernel scaffold; baseline (speedup 1.0000x reference)
import jax
import jax.numpy as jnp
from jax.experimental import pallas as pl


def kernel(edge_index, emb_users, emb_items):
    raise NotImplementedError("write your pallas kernel here")


# speedup vs baseline: 8.3250x; 8.3250x over previous
import functools

import jax
import jax.numpy as jnp
from jax import lax
from jax.experimental import pallas as pl
from jax.experimental.pallas import tpu as pltpu
from jax.experimental.pallas import tpu_sc as plsc

N_USERS = 25000
N_ITEMS = 25000
N_NODES = N_USERS + N_ITEMS
D = 64
E = 800000
LAYERS = 4

NC = 2
NS = 16
HALF = 25088
NPAD = 2 * HALF
ACC_ROWS = 25600
ZONE = ACC_ROWS // NS
R_T = HALF // NS
RC = 112
NRC = R_T // RC
E_T = 50176
CE = 128
NCH = E_T // CE
E_PAD = NS * E_T
PAD_DST = 1 << 29

_mesh = plsc.VectorSubcoreMesh(core_axis_name="c", subcore_axis_name="s")


def _rsqrt16(d):
    i = lax.bitcast_convert_type(d, jnp.int32)
    y = lax.bitcast_convert_type(jnp.int32(0x5F3759DF) - (i >> 1), jnp.float32)
    for _ in range(3):
        y = y * (1.5 - 0.5 * d * y * y)
    return y


def _redirect(dstv_ref, dsti_ref, c, dummy):
    base = c * HALF
    for k in range(CE // 16):
        dv = dstv_ref[pl.ds(k * 16, 16)]
        ld = dv - base
        ok = (ld >= 0) & (ld < HALF)
        dsti_ref[pl.ds(k * 16, 16)] = jnp.where(ok, ld, dummy)


@functools.partial(
    pl.kernel,
    out_type=(
        jax.ShapeDtypeStruct((NPAD,), jnp.float32),
        jax.ShapeDtypeStruct((NPAD, D), jnp.float32),
    ),
    mesh=_mesh,
    compiler_params=pltpu.CompilerParams(needs_layout_passes=False, use_tc_tiling_on_sc=False),
    scratch_types=dict(
        deg_sh=pltpu.VMEM_SHARED((ACC_ROWS,), jnp.float32),
        dstv=pltpu.VMEM((CE,), jnp.int32),
        dsti=pltpu.VMEM((CE,), jnp.int32),
        onesv=pltpu.VMEM((CE,), jnp.float32),
        zb1=pltpu.VMEM((ZONE,), jnp.float32),
        degv=pltpu.VMEM((RC,), jnp.float32),
        disv=pltpu.VMEM((RC,), jnp.float32),
        rb=pltpu.VMEM((RC, D), jnp.float32),
    ),
)
def _deg_dis_z0(dst_hbm, x0_hbm, zeros1d_hbm, ones_hbm,
                dis_out, z0_out,
                deg_sh, dstv, dsti, onesv, zb1, degv, disv, rb):
    c = lax.axis_index("c")
    s = lax.axis_index("s")
    dummy = HALF + s

    pltpu.sync_copy(zeros1d_hbm, zb1)
    pltpu.sync_copy(zb1, deg_sh.at[pl.ds(s * ZONE, ZONE)])
    pltpu.sync_copy(ones_hbm, onesv)
    plsc.subcore_barrier()

    ebase = s * E_T

    @pl.loop(0, NCH)
    def _deg_loop(g):
        pltpu.sync_copy(dst_hbm.at[pl.ds(ebase + g * CE, CE)], dstv)
        _redirect(dstv, dsti, c, dummy)
        pltpu.sync_copy(onesv, deg_sh.at[dsti], add=True)

    plsc.subcore_barrier()

    for j in range(NRC):
        lbase = s * R_T + j * RC
        gbase = c * HALF + lbase
        pltpu.sync_copy(deg_sh.at[pl.ds(lbase, RC)], degv)
        for k in range(RC // 16):
            dv = degv[pl.ds(k * 16, 16)]
            disv[pl.ds(k * 16, 16)] = jnp.where(dv > 0.0, _rsqrt16(dv), 0.0)
        pltpu.sync_copy(disv, dis_out.at[pl.ds(gbase, RC)])
        pltpu.sync_copy(x0_hbm.at[pl.ds(gbase, RC)], rb)

        @pl.loop(0, RC)
        def _scale_loop(r):
            b = plsc.load_gather(disv, [jnp.full((16,), r, jnp.int32)])
            for q in range(D // 16):
                rb[r, pl.ds(q * 16, 16)] = rb[r, pl.ds(q * 16, 16)] * b

        pltpu.sync_copy(rb, z0_out.at[pl.ds(gbase, RC)])


def _make_layer(s_scale, want_z):
    out_type = [jax.ShapeDtypeStruct((NPAD, D), jnp.float32)]
    if want_z:
        out_type.append(jax.ShapeDtypeStruct((NPAD, D), jnp.float32))

    @functools.partial(
        pl.kernel,
        out_type=tuple(out_type),
        mesh=_mesh,
        compiler_params=pltpu.CompilerParams(needs_layout_passes=False, use_tc_tiling_on_sc=False),
        scratch_types=dict(
            acc=pltpu.VMEM_SHARED((ACC_ROWS, D), jnp.float32),
            srcv=[pltpu.VMEM((CE,), jnp.int32) for _ in range(2)],
            dstv=pltpu.VMEM((CE,), jnp.int32),
            dsti=[pltpu.VMEM((CE,), jnp.int32) for _ in range(2)],
            rows=[pltpu.VMEM((CE, D), jnp.float32) for _ in range(2)],
            sems=[pltpu.SemaphoreType.DMA for _ in range(2)],
            disv=pltpu.VMEM((RC,), jnp.float32),
        ),
    )
    def _layer(src_hbm, dst_hbm, dis_hbm, z_hbm, s_hbm, zeros2d_hbm,
               *outs, acc, srcv, dstv, dsti, rows, sems, disv):
        rb = rows[1].at[pl.ds(0, RC)]
        sb = rows[0].at[pl.ds(0, RC)]
        s_out = outs[0]
        z_out = outs[1] if want_z else None

        c = lax.axis_index("c")
        s = lax.axis_index("s")
        dummy = HALF + s
        ebase = s * E_T

        pltpu.sync_copy(zeros2d_hbm, rows[0])
        for u in range(ZONE // CE):
            pltpu.sync_copy(rows[0], acc.at[pl.ds(s * ZONE + u * CE, CE)])
        pltpu.sync_copy(rows[0].at[pl.ds(0, ZONE - (ZONE // CE) * CE)],
                        acc.at[pl.ds(s * ZONE + (ZONE // CE) * CE,
                                     ZONE - (ZONE // CE) * CE)])
        plsc.subcore_barrier()

        def _load_chunk(ci, b):
            pltpu.sync_copy(src_hbm.at[pl.ds(ebase + ci * CE, CE)], srcv[b])
            pltpu.sync_copy(dst_hbm.at[pl.ds(ebase + ci * CE, CE)], dstv)
            _redirect(dstv, dsti[b], c, dummy)
            pltpu.async_copy(z_hbm.at[srcv[b]], rows[b], sems[b])

        _load_chunk(0, 0)

        @pl.loop(0, NCH, step=2)
        def _edge_loop(g):
            for b in range(2):
                ci = g + b

                @pl.when(ci + 1 < NCH)
                def _():
                    _load_chunk(ci + 1, 1 - b)

                pltpu.make_async_copy(z_hbm.at[srcv[b]], rows[b], sems[b]).wait()
                pltpu.sync_copy(rows[b], acc.at[dsti[b]], add=True)

        plsc.subcore_barrier()

        for j in range(NRC):
            lbase = s * R_T + j * RC
            gbase = c * HALF + lbase
            pltpu.sync_copy(acc.at[pl.ds(lbase, RC)], rb)
            pltpu.sync_copy(dis_hbm.at[pl.ds(gbase, RC)], disv)
            pltpu.sync_copy(s_hbm.at[pl.ds(gbase, RC)], sb)

            @pl.loop(0, RC)
            def _scale_loop(r):
                bv = plsc.load_gather(disv, [jnp.full((16,), r, jnp.int32)])
                for q in range(D // 16):
                    sl = pl.ds(q * 16, 16)
                    x = rb[r, sl] * bv
                    sb[r, sl] = (sb[r, sl] + x) * s_scale
                    if want_z:
                        rb[r, sl] = x * bv

            pltpu.sync_copy(sb, s_out.at[pl.ds(gbase, RC)])
            if want_z:
                pltpu.sync_copy(rb, z_out.at[pl.ds(gbase, RC)])

    return _layer


_layer_mid = _make_layer(1.0, True)
_layer_last = _make_layer(1.0 / ((LAYERS + 1.0) ** 2), False)


def kernel(edge_index, emb_users, emb_items):
    src = edge_index[0]
    dst = edge_index[1]
    pad_e = E_PAD - E
    src_p = jnp.concatenate([src, jnp.zeros((pad_e,), jnp.int32)])
    dst_p = jnp.concatenate([dst, jnp.full((pad_e,), PAD_DST, jnp.int32)])
    x0 = jnp.concatenate(
        [emb_users, emb_items, jnp.zeros((NPAD - N_NODES, D), jnp.float32)], axis=0
    )
    zeros1d = jnp.zeros((ZONE,), jnp.float32)
    zeros2d = jnp.zeros((CE, D), jnp.float32)
    ones = jnp.ones((CE,), jnp.float32)

    dis, z = _deg_dis_z0(dst_p, x0, zeros1d, ones)
    S = x0
    for k in range(LAYERS):
        if k < LAYERS - 1:
            S, z = _layer_mid(src_p, dst_p, dis, z, S, zeros2d)
        else:
            (S,) = _layer_last(src_p, dst_p, dis, z, S, zeros2d)

    emb_users_final = S[:N_USERS]
    emb_items_final = S[N_USERS:N_NODES]
    return (emb_users_final, emb_users, emb_items_final, emb_items)


# speedup vs baseline: 10.7552x; 1.2919x over previous
import functools

import jax
import jax.numpy as jnp
from jax import lax
from jax.experimental import pallas as pl
from jax.experimental.pallas import tpu as pltpu
from jax.experimental.pallas import tpu_sc as plsc

N_USERS = 25000
N_ITEMS = 25000
N_NODES = N_USERS + N_ITEMS
D = 64
E = 800000
LAYERS = 4

NC = 2
NS = 16
HALF = 25088
NPAD = 2 * HALF
ACC_ROWS = 25216
ZONE = ACC_ROWS // NS
R_T = HALF // NS
RC = 112
NRC = R_T // RC
E_T = 50176
CE = 128
NCH = E_T // CE
BLK = 8
NB = NCH // BLK
E_PAD = NS * E_T
PAD_DST = 1 << 29

_params = pltpu.CompilerParams(needs_layout_passes=False, use_tc_tiling_on_sc=False)
_mesh = plsc.VectorSubcoreMesh(core_axis_name="c", subcore_axis_name="s")


def _rsqrt16(d):
    i = lax.bitcast_convert_type(d, jnp.int32)
    y = lax.bitcast_convert_type(jnp.int32(0x5F3759DF) - (i >> 1), jnp.float32)
    for _ in range(3):
        y = y * (1.5 - 0.5 * d * y * y)
    return y


@functools.partial(
    pl.kernel,
    out_type=(
        jax.ShapeDtypeStruct((NPAD,), jnp.float32),
        jax.ShapeDtypeStruct((NPAD, D), jnp.float32),
        jax.ShapeDtypeStruct((NC, NS, NCH, CE), jnp.int32),
    ),
    mesh=_mesh,
    compiler_params=_params,
    scratch_types=dict(
        deg_sh=pltpu.VMEM_SHARED((ACC_ROWS,), jnp.float32),
        dst_blk=pltpu.VMEM((BLK, CE), jnp.int32),
        dsti_blk=[pltpu.VMEM((BLK, CE), jnp.int32) for _ in range(2)],
        onesv=pltpu.VMEM((CE,), jnp.float32),
        zb1=pltpu.VMEM((ZONE,), jnp.float32),
        degv=pltpu.VMEM((RC,), jnp.float32),
        disv=pltpu.VMEM((RC,), jnp.float32),
        rb=pltpu.VMEM((RC, D), jnp.float32),
    ),
)
def _deg_dis_z0(dst_hbm, x0_hbm, zeros1d_hbm, ones_hbm,
                dis_out, z0_out, dsti_out,
                deg_sh, dst_blk, dsti_blk, onesv, zb1, degv, disv, rb):
    c = lax.axis_index("c")
    s = lax.axis_index("s")
    dummy = HALF + s
    base = c * HALF

    pltpu.sync_copy(zeros1d_hbm, zb1)
    pltpu.sync_copy(zb1, deg_sh.at[pl.ds(s * ZONE, ZONE)])
    pltpu.sync_copy(ones_hbm, onesv)
    plsc.subcore_barrier()

    def _deg_block(blk, p):
        pltpu.sync_copy(dst_hbm.at[s, pl.ds(blk * BLK, BLK)], dst_blk)
        for j in range(BLK):
            for k in range(CE // 16):
                dv = dst_blk[j, pl.ds(k * 16, 16)]
                ld = dv - base
                ok = (ld >= 0) & (ld < HALF)
                dsti_blk[p][j, pl.ds(k * 16, 16)] = jnp.where(ok, ld, dummy)
        pltpu.sync_copy(dsti_blk[p], dsti_out.at[c, s, pl.ds(blk * BLK, BLK)])
        for j in range(BLK):
            pltpu.sync_copy(onesv, deg_sh.at[dsti_blk[p].at[j]], add=True)

    @pl.loop(0, NB - 1, step=2)
    def _deg_blocks(g2):
        for p in range(2):
            _deg_block(g2 + p, p)

    _deg_block(NB - 1, 0)
    plsc.subcore_barrier()

    for j in range(NRC):
        lbase = s * R_T + j * RC
        gbase = c * HALF + lbase
        pltpu.sync_copy(deg_sh.at[pl.ds(lbase, RC)], degv)
        for k in range(RC // 16):
            dv = degv[pl.ds(k * 16, 16)]
            disv[pl.ds(k * 16, 16)] = jnp.where(dv > 0.0, _rsqrt16(dv), 0.0)
        pltpu.sync_copy(disv, dis_out.at[pl.ds(gbase, RC)])
        pltpu.sync_copy(x0_hbm.at[pl.ds(gbase, RC)], rb)

        @pl.loop(0, RC)
        def _scale_loop(r):
            b = plsc.load_gather(disv, [jnp.full((16,), r, jnp.int32)])
            for q in range(D // 16):
                rb[r, pl.ds(q * 16, 16)] = rb[r, pl.ds(q * 16, 16)] * b

        pltpu.sync_copy(rb, z0_out.at[pl.ds(gbase, RC)])


def _make_layer(s_scale, want_z):
    out_type = [jax.ShapeDtypeStruct((NPAD, D), jnp.float32)]
    if want_z:
        out_type.append(jax.ShapeDtypeStruct((NPAD, D), jnp.float32))

    @functools.partial(
        pl.kernel,
        out_type=tuple(out_type),
        mesh=_mesh,
        compiler_params=_params,
        scratch_types=dict(
            acc=pltpu.VMEM_SHARED((ACC_ROWS, D), jnp.float32),
            src_blk=[pltpu.VMEM((BLK, CE), jnp.int32) for _ in range(2)],
            dsti_blk=[pltpu.VMEM((BLK, CE), jnp.int32) for _ in range(2)],
            rows=[pltpu.VMEM((CE, D), jnp.float32) for _ in range(2)],
            gsem=[pltpu.SemaphoreType.DMA for _ in range(2)],
            disv=pltpu.VMEM((RC,), jnp.float32),
        ),
    )
    def _layer(src_hbm, dsti_hbm, dis_hbm, z_hbm, s_hbm, zeros2d_hbm,
               *outs, acc, src_blk, dsti_blk, rows, gsem, disv):
        s_out = outs[0]
        z_out = outs[1] if want_z else None
        rb = rows[1].at[pl.ds(0, RC)]
        sb = rows[0].at[pl.ds(0, RC)]

        c = lax.axis_index("c")
        s = lax.axis_index("s")

        pltpu.sync_copy(zeros2d_hbm, rows[0])
        for u in range(ZONE // CE):
            pltpu.sync_copy(rows[0], acc.at[pl.ds(s * ZONE + u * CE, CE)])
        rem = ZONE - (ZONE // CE) * CE
        pltpu.sync_copy(rows[0].at[pl.ds(0, rem)],
                        acc.at[pl.ds(s * ZONE + (ZONE // CE) * CE, rem)])
        plsc.subcore_barrier()

        def _gather(p, j, b):
            pltpu.async_copy(z_hbm.at[src_blk[p].at[j]], rows[b], gsem[b])

        def _wait_gather(b):
            pltpu.make_async_copy(z_hbm.at[src_blk[0].at[0]], rows[b], gsem[b]).wait()

        def _scatter(p, j, b):
            pltpu.sync_copy(rows[b], acc.at[dsti_blk[p].at[j]], add=True)

        def _edge_block(blk, p):
            pltpu.sync_copy(src_hbm.at[s, pl.ds(blk * BLK, BLK)], src_blk[p])
            pltpu.sync_copy(dsti_hbm.at[c, s, pl.ds(blk * BLK, BLK)],
                            dsti_blk[p])
            _gather(p, 0, 0)
            for j in range(BLK):
                b = j % 2
                if j + 1 < BLK:
                    _gather(p, j + 1, (j + 1) % 2)
                _wait_gather(b)
                _scatter(p, j, b)

        @pl.loop(0, NB - 1, step=2)
        def _edge_blocks(g2):
            for p in range(2):
                _edge_block(g2 + p, p)

        _edge_block(NB - 1, 0)
        plsc.subcore_barrier()

        for j in range(NRC):
            lbase = s * R_T + j * RC
            gbase = c * HALF + lbase
            pltpu.sync_copy(acc.at[pl.ds(lbase, RC)], rb)
            pltpu.sync_copy(dis_hbm.at[pl.ds(gbase, RC)], disv)
            pltpu.sync_copy(s_hbm.at[pl.ds(gbase, RC)], sb)

            @pl.loop(0, RC)
            def _scale_loop(r):
                bv = plsc.load_gather(disv, [jnp.full((16,), r, jnp.int32)])
                for q in range(D // 16):
                    sl = pl.ds(q * 16, 16)
                    x = rb[r, sl] * bv
                    sb[r, sl] = (sb[r, sl] + x) * s_scale
                    if want_z:
                        rb[r, sl] = x * bv

            pltpu.sync_copy(sb, s_out.at[pl.ds(gbase, RC)])
            if want_z:
                pltpu.sync_copy(rb, z_out.at[pl.ds(gbase, RC)])

    return _layer


_layer_mid = _make_layer(1.0, True)
_layer_last = _make_layer(1.0 / ((LAYERS + 1.0) ** 2), False)


def kernel(edge_index, emb_users, emb_items):
    src = edge_index[0]
    dst = edge_index[1]
    pad_e = E_PAD - E
    src_r = jnp.concatenate(
        [src, jnp.zeros((pad_e,), jnp.int32)]).reshape(NS, NCH, CE)
    dst_r = jnp.concatenate(
        [dst, jnp.full((pad_e,), PAD_DST, jnp.int32)]).reshape(NS, NCH, CE)
    x0 = jnp.concatenate(
        [emb_users, emb_items, jnp.zeros((NPAD - N_NODES, D), jnp.float32)], axis=0
    )
    zeros1d = jnp.zeros((ZONE,), jnp.float32)
    zeros2d = jnp.zeros((CE, D), jnp.float32)
    ones = jnp.ones((CE,), jnp.float32)

    dis, z, dsti = _deg_dis_z0(dst_r, x0, zeros1d, ones)
    S = x0
    for k in range(LAYERS):
        if k < LAYERS - 1:
            S, z = _layer_mid(src_r, dsti, dis, z, S, zeros2d)
        else:
            (S,) = _layer_last(src_r, dsti, dis, z, S, zeros2d)

    emb_users_final = S[:N_USERS]
    emb_items_final = S[N_USERS:N_NODES]
    return (emb_users_final, emb_users, emb_items_final, emb_items)


# speedup vs baseline: 11.0451x; 1.0270x over previous
import functools

import jax
import jax.numpy as jnp
from jax import lax
from jax.experimental import pallas as pl
from jax.experimental.pallas import tpu as pltpu
from jax.experimental.pallas import tpu_sc as plsc

N_USERS = 25000
N_ITEMS = 25000
N_NODES = N_USERS + N_ITEMS
D = 64
E = 800000
LAYERS = 4

NC = 2
NS = 16
HALF = 25088
NPAD = 2 * HALF
ACC_ROWS = 25216
ZONE = ACC_ROWS // NS
R_T = HALF // NS
RC = 112
NRC = R_T // RC
E_T = 50176
CE = 128
NCH = E_T // CE
BLK = 8
NB = NCH // BLK
E_PAD = NS * E_T
PAD_DST = 1 << 29

_params = pltpu.CompilerParams(needs_layout_passes=False, use_tc_tiling_on_sc=False)
_mesh = plsc.VectorSubcoreMesh(core_axis_name="c", subcore_axis_name="s")


def _rsqrt16(d):
    i = lax.bitcast_convert_type(d, jnp.int32)
    y = lax.bitcast_convert_type(jnp.int32(0x5F3759DF) - (i >> 1), jnp.float32)
    for _ in range(3):
        y = y * (1.5 - 0.5 * d * y * y)
    return y


@functools.partial(
    pl.kernel,
    out_type=(
        jax.ShapeDtypeStruct((NPAD,), jnp.float32),
        jax.ShapeDtypeStruct((NPAD, D), jnp.float32),
        jax.ShapeDtypeStruct((NC, NS, NCH, CE), jnp.int32),
    ),
    mesh=_mesh,
    compiler_params=_params,
    scratch_types=dict(
        deg_sh=pltpu.VMEM_SHARED((ACC_ROWS,), jnp.float32),
        dst_blk=pltpu.VMEM((BLK, CE), jnp.int32),
        dsti_blk=[pltpu.VMEM((BLK, CE), jnp.int32) for _ in range(2)],
        onesv=pltpu.VMEM((CE,), jnp.float32),
        zb1=pltpu.VMEM((ZONE,), jnp.float32),
        degv=pltpu.VMEM((RC,), jnp.float32),
        disv=pltpu.VMEM((RC,), jnp.float32),
        rb=pltpu.VMEM((RC, D), jnp.float32),
    ),
)
def _deg_dis_z0(dst_hbm, x0_hbm, zeros1d_hbm, ones_hbm,
                dis_out, z0_out, dsti_out,
                deg_sh, dst_blk, dsti_blk, onesv, zb1, degv, disv, rb):
    c = lax.axis_index("c")
    s = lax.axis_index("s")
    dummy = HALF + s
    base = c * HALF

    pltpu.sync_copy(zeros1d_hbm, zb1)
    pltpu.sync_copy(zb1, deg_sh.at[pl.ds(s * ZONE, ZONE)])
    pltpu.sync_copy(ones_hbm, onesv)
    plsc.subcore_barrier()

    def _deg_block(blk, p):
        pltpu.sync_copy(dst_hbm.at[s, pl.ds(blk * BLK, BLK)], dst_blk)
        for j in range(BLK):
            for k in range(CE // 16):
                dv = dst_blk[j, pl.ds(k * 16, 16)]
                ld = dv - base
                ok = (ld >= 0) & (ld < HALF)
                dsti_blk[p][j, pl.ds(k * 16, 16)] = jnp.where(ok, ld, dummy)
        pltpu.sync_copy(dsti_blk[p], dsti_out.at[c, s, pl.ds(blk * BLK, BLK)])
        for j in range(BLK):
            pltpu.sync_copy(onesv, deg_sh.at[dsti_blk[p].at[j]], add=True)

    @pl.loop(0, NB - 1, step=2)
    def _deg_blocks(g2):
        for p in range(2):
            _deg_block(g2 + p, p)

    _deg_block(NB - 1, 0)
    plsc.subcore_barrier()

    for j in range(NRC):
        lbase = s * R_T + j * RC
        gbase = c * HALF + lbase
        pltpu.sync_copy(deg_sh.at[pl.ds(lbase, RC)], degv)
        for k in range(RC // 16):
            dv = degv[pl.ds(k * 16, 16)]
            disv[pl.ds(k * 16, 16)] = jnp.where(dv > 0.0, _rsqrt16(dv), 0.0)
        pltpu.sync_copy(disv, dis_out.at[pl.ds(gbase, RC)])
        pltpu.sync_copy(x0_hbm.at[pl.ds(gbase, RC)], rb)

        @pl.loop(0, RC)
        def _scale_loop(r):
            b = plsc.load_gather(disv, [jnp.full((16,), r, jnp.int32)])
            for q in range(D // 16):
                rb[r, pl.ds(q * 16, 16)] = rb[r, pl.ds(q * 16, 16)] * b

        pltpu.sync_copy(rb, z0_out.at[pl.ds(gbase, RC)])


def _make_layer(s_scale, want_z):
    out_type = [jax.ShapeDtypeStruct((NPAD, D), jnp.float32)]
    if want_z:
        out_type.append(jax.ShapeDtypeStruct((NPAD, D), jnp.float32))

    @functools.partial(
        pl.kernel,
        out_type=tuple(out_type),
        mesh=_mesh,
        compiler_params=_params,
        scratch_types=dict(
            acc=pltpu.VMEM_SHARED((ACC_ROWS, D), jnp.float32),
            src_blk=[pltpu.VMEM((BLK, CE), jnp.int32) for _ in range(2)],
            dsti_blk=[pltpu.VMEM((BLK, CE), jnp.int32) for _ in range(2)],
            rows=[pltpu.VMEM((CE, D), jnp.float32) for _ in range(2)],
            gsem=[pltpu.SemaphoreType.DMA for _ in range(2)],
            ssem=[pltpu.SemaphoreType.DMA for _ in range(2)],
            disv=pltpu.VMEM((RC,), jnp.float32),
        ),
    )
    def _layer(src_hbm, dsti_hbm, dis_hbm, z_hbm, s_hbm, zeros2d_hbm,
               *outs, acc, src_blk, dsti_blk, rows, gsem, ssem, disv):
        s_out = outs[0]
        z_out = outs[1] if want_z else None
        rb = rows[1].at[pl.ds(0, RC)]
        sb = rows[0].at[pl.ds(0, RC)]

        c = lax.axis_index("c")
        s = lax.axis_index("s")

        pltpu.sync_copy(zeros2d_hbm, rows[0])
        for u in range(ZONE // CE):
            pltpu.sync_copy(rows[0], acc.at[pl.ds(s * ZONE + u * CE, CE)])
        rem = ZONE - (ZONE // CE) * CE
        pltpu.sync_copy(rows[0].at[pl.ds(0, rem)],
                        acc.at[pl.ds(s * ZONE + (ZONE // CE) * CE, rem)])
        plsc.subcore_barrier()

        def _gather(p, j, b):
            pltpu.async_copy(z_hbm.at[src_blk[p].at[j]], rows[b], gsem[b])

        def _wait_gather(b):
            pltpu.make_async_copy(z_hbm.at[src_blk[0].at[0]], rows[b], gsem[b]).wait()

        def _scatter(p, j, b):
            pltpu.async_copy(rows[b], acc.at[dsti_blk[p].at[j]], ssem[b], add=True)

        def _wait_scatter(b, guard=None):
            def _w():
                pltpu.make_async_copy(rows[b], acc.at[dsti_blk[0].at[0]],
                                      ssem[b]).wait()
            if guard is None:
                _w()
            else:
                pl.when(guard)(_w)

        def _edge_block(blk, p, wait_cond):
            pltpu.sync_copy(src_hbm.at[s, pl.ds(blk * BLK, BLK)], src_blk[p])
            pltpu.sync_copy(dsti_hbm.at[c, s, pl.ds(blk * BLK, BLK)],
                            dsti_blk[p])
            _wait_scatter(0, wait_cond)
            _gather(p, 0, 0)
            for j in range(BLK):
                b = j % 2
                if j + 1 < BLK:
                    nb = (j + 1) % 2
                    _wait_scatter(nb, wait_cond if j == 0 else None)
                    _gather(p, j + 1, nb)
                _wait_gather(b)
                _scatter(p, j, b)

        @pl.loop(0, NB - 1, step=2)
        def _edge_blocks(g2):
            _edge_block(g2, 0, g2 > 0)
            _edge_block(g2 + 1, 1, None)

        _edge_block(NB - 1, 0, None)
        _wait_scatter(0)
        _wait_scatter(1)
        plsc.subcore_barrier()

        for j in range(NRC):
            lbase = s * R_T + j * RC
            gbase = c * HALF + lbase
            pltpu.sync_copy(acc.at[pl.ds(lbase, RC)], rb)
            pltpu.sync_copy(dis_hbm.at[pl.ds(gbase, RC)], disv)
            pltpu.sync_copy(s_hbm.at[pl.ds(gbase, RC)], sb)

            @pl.loop(0, RC)
            def _scale_loop(r):
                bv = plsc.load_gather(disv, [jnp.full((16,), r, jnp.int32)])
                for q in range(D // 16):
                    sl = pl.ds(q * 16, 16)
                    x = rb[r, sl] * bv
                    sb[r, sl] = (sb[r, sl] + x) * s_scale
                    if want_z:
                        rb[r, sl] = x * bv

            pltpu.sync_copy(sb, s_out.at[pl.ds(gbase, RC)])
            if want_z:
                pltpu.sync_copy(rb, z_out.at[pl.ds(gbase, RC)])

    return _layer


_layer_mid = _make_layer(1.0, True)
_layer_last = _make_layer(1.0 / ((LAYERS + 1.0) ** 2), False)


def kernel(edge_index, emb_users, emb_items):
    src = edge_index[0]
    dst = edge_index[1]
    pad_e = E_PAD - E
    src_r = jnp.concatenate(
        [src, jnp.zeros((pad_e,), jnp.int32)]).reshape(NS, NCH, CE)
    dst_r = jnp.concatenate(
        [dst, jnp.full((pad_e,), PAD_DST, jnp.int32)]).reshape(NS, NCH, CE)
    x0 = jnp.concatenate(
        [emb_users, emb_items, jnp.zeros((NPAD - N_NODES, D), jnp.float32)], axis=0
    )
    zeros1d = jnp.zeros((ZONE,), jnp.float32)
    zeros2d = jnp.zeros((CE, D), jnp.float32)
    ones = jnp.ones((CE,), jnp.float32)

    dis, z, dsti = _deg_dis_z0(dst_r, x0, zeros1d, ones)
    S = x0
    for k in range(LAYERS):
        if k < LAYERS - 1:
            S, z = _layer_mid(src_r, dsti, dis, z, S, zeros2d)
        else:
            (S,) = _layer_last(src_r, dsti, dis, z, S, zeros2d)

    emb_users_final = S[:N_USERS]
    emb_items_final = S[N_USERS:N_NODES]
    return (emb_users_final, emb_users, emb_items_final, emb_items)
